# baseline (speedup 1.0000x reference)
import jax
import jax.numpy as jnp
from jax import lax
from jax.experimental import pallas as pl
from jax.experimental.pallas import tpu as pltpu
from jax.experimental.pallas import tpu_sc as plsc

N = 10000
N_PAD = 10240
E = 320000
D = 128
ALPHA = 0.15
N_PROP = 5

NC = 2
NS = 16
L = 16

DH = D // NC
EPT = 20480
E_PAD = NS * EPT
CB = 128
KCH = EPT // CB
NBUF = 4
RPT = N_PAD // NS
RB = 128
NBLK = RPT // RB
FV = DH // L


def _lane_bcast(v, j):
  return lax.gather(
      v, jnp.full((L, 1), j, jnp.int32),
      lax.GatherDimensionNumbers(
          offset_dims=(), collapsed_slice_dims=(0,), start_index_map=(0,)),
      (1,), mode=lax.GatherScatterMode.PROMISE_IN_BOUNDS)


def _body(logits_hbm, h0s_hbm, pack_hbm, w_hbm, out_hbm,
          ring, wring, sbuf, rows0, rows1, rows2, rows3,
          frows0, frows1, frows2, frows3, h_sp, acc_sp,
          r0s, r1s, r2s, r3s, g0, g1, g2, g3, s0, s1, s2, s3):
  cid = lax.axis_index("c")
  sid = lax.axis_index("s")
  r0 = sid * RPT
  bufs = [rows0, rows1, rows2, rows3]
  fbufs = [frows0, frows1, frows2, frows3]
  rsems = [r0s, r1s, r2s, r3s]
  gsems = [g0, g1, g2, g3]
  ssems = [s0, s1, s2, s3]

  def zero_buf(buf):
    def zbody(r, carry):
      for f in range(FV):
        buf[r, pl.ds(f * L, L)] = jnp.zeros((L,), jnp.float32)
      return carry
    lax.fori_loop(0, RB, zbody, 0)

  def rstart(k, sl):
    pltpu.async_copy(pack_hbm.at[sid, k], ring.at[sl], rsems[sl])
    pltpu.async_copy(w_hbm.at[sid, k], wring.at[sl], rsems[sl])

  def rwait(k, sl):
    pltpu.make_async_copy(pack_hbm.at[sid, k], ring.at[sl], rsems[sl]).wait()
    pltpu.make_async_copy(w_hbm.at[sid, k], wring.at[sl], rsems[sl]).wait()

  def gstart(sl):
    pltpu.async_copy(h_sp.at[ring.at[sl, 0]], bufs[sl], gsems[sl])

  def gwait(sl):
    pltpu.make_async_copy(h_sp.at[ring.at[sl, 0]], bufs[sl], gsems[sl]).wait()

  def sstart(sl):
    pltpu.async_copy(fbufs[sl], acc_sp.at[sbuf.at[sl]], ssems[sl], add=True)

  def swait(sl):
    pltpu.make_async_copy(fbufs[sl], acc_sp.at[sbuf.at[sl]], ssems[sl]).wait()

  def scale(b):
    buf = bufs[b]
    fbuf = fbufs[b]
    for g in range(CB // L):
      sbuf[b, pl.ds(g * L, L)] = ring[b, 1, pl.ds(g * L, L)]
      w16 = wring[b, pl.ds(g * L, L)]
      for j in range(L):
        e = g * L + j
        we = _lane_bcast(w16, j)
        for h in range(FV // 2):
          v32 = buf[e, pl.ds(h * 2 * L, 2 * L)]
          va, vb = plsc.unpack(v32, format=plsc.PackFormat.INTERLEAVED)
          fbuf[e, pl.ds(h * 2 * L, L)] = va * we
          fbuf[e, pl.ds(h * 2 * L + L, L)] = vb * we

  def pack_rows(fbuf, gbuf):
    def pbody(r, carry):
      for h in range(FV // 2):
        va = fbuf[r, pl.ds(h * 2 * L, L)]
        vb = fbuf[r, pl.ds(h * 2 * L + L, L)]
        gbuf[r, pl.ds(h * 2 * L, 2 * L)] = plsc.pack(
            va, vb, format=plsc.PackFormat.INTERLEAVED)
      return carry
    lax.fori_loop(0, RB, pbody, 0)

  for b in range(NBLK):
    pltpu.sync_copy(logits_hbm.at[cid, pl.ds(r0 + b * RB, RB)], frows0)
    pack_rows(frows0, rows0)
    pltpu.sync_copy(rows0, h_sp.at[pl.ds(r0 + b * RB, RB)])
    pltpu.sync_copy(h0s_hbm.at[cid, pl.ds(r0 + b * RB, RB)],
                    acc_sp.at[pl.ds(r0 + b * RB, RB)])

  plsc.subcore_barrier()

  def iteration(_, carry):
    for i in range(NBUF - 1):
      rstart(i, i)
    for i in range(2):
      rwait(i, i)
      gstart(i)

    def edge_quad(kk, c2):
      for b in range(NBUF):
        k = NBUF * kk + b
        kp = k + 3
        slp = (b + 3) % NBUF
        if b == 0:
          rstart(kp, slp)
        else:
          @pl.when(kp < KCH)
          def _():
            rstart(kp, slp)

        kg = k + 2
        slg = (b + 2) % NBUF
        if b < 2:
          @pl.when(kk > 0)
          def _():
            swait(slg)
          rwait(kg, slg)
          gstart(slg)
        else:
          @pl.when(kg < KCH)
          def _():
            swait(slg)
            rwait(kg, slg)
            gstart(slg)

        gwait(b)
        scale(b)
        sstart(b)
      return c2

    lax.fori_loop(0, KCH // NBUF, edge_quad, 0)
    for b in range(NBUF):
      swait(b)
    plsc.subcore_barrier()

    for b in range(NBLK):
      rs = r0 + b * RB
      pltpu.sync_copy(acc_sp.at[pl.ds(rs, RB)], frows0)
      pack_rows(frows0, rows0)
      pltpu.sync_copy(rows0, h_sp.at[pl.ds(rs, RB)])
      pltpu.sync_copy(frows0, out_hbm.at[cid, pl.ds(rs, RB)])
      pltpu.sync_copy(h0s_hbm.at[cid, pl.ds(rs, RB)],
                      acc_sp.at[pl.ds(rs, RB)])

    plsc.subcore_barrier()
    return carry

  lax.fori_loop(0, N_PROP, iteration, 0)


@jax.jit
def _pagerank(logits_split, h0s, pack, w):
  mesh = plsc.VectorSubcoreMesh(
      core_axis_name="c", subcore_axis_name="s", num_cores=NC, num_subcores=NS)
  return pl.kernel(
      _body,
      out_type=jax.ShapeDtypeStruct((NC, N_PAD, DH), jnp.float32),
      mesh=mesh,
      compiler_params=pltpu.CompilerParams(use_tc_tiling_on_sc=False, needs_layout_passes=False),
      scratch_types=[
          pltpu.VMEM((NBUF, 2, CB), jnp.int32),
          pltpu.VMEM((NBUF, CB), jnp.float32),
          pltpu.VMEM((NBUF, CB), jnp.int32),
          pltpu.VMEM((CB, DH), jnp.bfloat16),
          pltpu.VMEM((CB, DH), jnp.bfloat16),
          pltpu.VMEM((CB, DH), jnp.bfloat16),
          pltpu.VMEM((CB, DH), jnp.bfloat16),
          pltpu.VMEM((CB, DH), jnp.float32),
          pltpu.VMEM((CB, DH), jnp.float32),
          pltpu.VMEM((CB, DH), jnp.float32),
          pltpu.VMEM((CB, DH), jnp.float32),
          pltpu.VMEM_SHARED((N_PAD, DH), jnp.bfloat16),
          pltpu.VMEM_SHARED((N_PAD, DH), jnp.float32),
      ] + [pltpu.SemaphoreType.DMA] * 12,
  )(logits_split, h0s, pack, w)


def kernel(logits, edge_index, edge_weight):
  dst = edge_index[0]
  src = edge_index[1]
  pad = E_PAD - E
  src_r = jnp.concatenate([src, jnp.zeros((pad,), jnp.int32)])
  dst_r = jnp.concatenate([dst, jnp.zeros((pad,), jnp.int32)])
  w_r = jnp.concatenate(
      [(1.0 - ALPHA) * edge_weight, jnp.zeros((pad,), jnp.float32)])
  pack = jnp.stack(
      [src_r.reshape(NS, KCH, CB), dst_r.reshape(NS, KCH, CB)], axis=2)
  w = w_r.reshape(NS, KCH, CB)
  logits_split = logits.reshape(N, NC, DH).transpose(1, 0, 2)
  logits_split = jnp.pad(logits_split, ((0, 0), (0, N_PAD - N), (0, 0)))
  perm32 = jnp.concatenate([jnp.arange(0, 2 * L, 2), jnp.arange(1, 2 * L, 2)])
  perm = jnp.concatenate([perm32 + 2 * L * i for i in range(DH // (2 * L))])
  logits_perm = logits_split[:, :, perm]
  out = _pagerank(logits_perm, ALPHA * logits_perm, pack, w)
  inv = jnp.argsort(perm)
  return out[:, :N, inv].transpose(1, 0, 2).reshape(N, D)


# speedup vs baseline: 1.0412x; 1.0412x over previous
import jax
import jax.numpy as jnp
from jax import lax
from jax.experimental import pallas as pl
from jax.experimental.pallas import tpu as pltpu
from jax.experimental.pallas import tpu_sc as plsc

N = 10000
N_PAD = 10240
E = 320000
D = 128
ALPHA = 0.15
N_PROP = 5

NC = 2
NS = 16
L = 16

DH = D // NC
EPT = 20480
E_PAD = NS * EPT
CB = 128
KCH = EPT // CB
NBUF = 4
NRS = 8
RPT = N_PAD // NS
RB = 128
NBLK = RPT // RB
FV = DH // L


def _lane_bcast(v, j):
  return lax.gather(
      v, jnp.full((L, 1), j, jnp.int32),
      lax.GatherDimensionNumbers(
          offset_dims=(), collapsed_slice_dims=(0,), start_index_map=(0,)),
      (1,), mode=lax.GatherScatterMode.PROMISE_IN_BOUNDS)


def _body(logits_hbm, h0s_hbm, pack_hbm, out_hbm,
          ring, rows0, rows1, rows2, rows3, h_sp, acc_sp,
          r0s, r1s, r2s, r3s, r4s, r5s, r6s, r7s,
          g0, g1, g2, g3, s0, s1, s2, s3):
  cid = lax.axis_index("c")
  sid = lax.axis_index("s")
  r0 = sid * RPT
  bufs = [rows0, rows1, rows2, rows3]
  rsems = [r0s, r1s, r2s, r3s, r4s, r5s, r6s, r7s]
  gsems = [g0, g1, g2, g3]
  ssems = [s0, s1, s2, s3]

  def zero_buf(buf):
    def zbody(r, carry):
      for f in range(FV):
        buf[r, pl.ds(f * L, L)] = jnp.zeros((L,), jnp.float32)
      return carry
    lax.fori_loop(0, RB, zbody, 0)

  def rstart(k, sl):
    pltpu.async_copy(pack_hbm.at[sid, k], ring.at[sl], rsems[sl])

  def rwait(k, sl):
    pltpu.make_async_copy(pack_hbm.at[sid, k], ring.at[sl], rsems[sl]).wait()

  def gstart(rs, b):
    pltpu.async_copy(h_sp.at[ring.at[rs, 0]], bufs[b], gsems[b])

  def gwait(rs, b):
    pltpu.make_async_copy(h_sp.at[ring.at[rs, 0]], bufs[b], gsems[b]).wait()

  def sstart(rs, b):
    pltpu.async_copy(bufs[b], acc_sp.at[ring.at[rs, 1]], ssems[b], add=True)

  def swait(rs, b):
    pltpu.make_async_copy(bufs[b], acc_sp.at[ring.at[rs, 1]], ssems[b]).wait()

  def scale(rs, b):
    buf = bufs[b]
    for g in range(CB // L):
      w16 = plsc.bitcast(ring[rs, 2, pl.ds(g * L, L)], jnp.float32)
      for j in range(L):
        e = g * L + j
        we = _lane_bcast(w16, j)
        for f in range(FV):
          buf[e, pl.ds(f * L, L)] = buf[e, pl.ds(f * L, L)] * we

  for b in range(NBLK):
    pltpu.sync_copy(logits_hbm.at[cid, pl.ds(r0 + b * RB, RB)], rows1)
    pltpu.sync_copy(rows1, h_sp.at[pl.ds(r0 + b * RB, RB)])
    pltpu.sync_copy(h0s_hbm.at[cid, pl.ds(r0 + b * RB, RB)],
                    acc_sp.at[pl.ds(r0 + b * RB, RB)])

  plsc.subcore_barrier()

  def iteration(_, carry):
    for i in range(3):
      rstart(i, i)
    for i in range(2):
      rwait(i, i)
      gstart(i, i)

    def edge_oct(kk, c2):
      for u in range(NRS):
        k = NRS * kk + u
        b = u % NBUF
        kp = k + 3
        slp = (u + 3) % NRS
        if u < 5:
          rstart(kp, slp)
        else:
          @pl.when(kp < KCH)
          def _():
            rstart(kp, slp)

        kg = k + 2
        bg = (u + 2) % NBUF
        slg = (u + 2) % NRS
        if u < 2:
          @pl.when(kk > 0)
          def _():
            swait((slg - 4) % NRS, bg)
          rwait(kg, slg)
          gstart(slg, bg)
        elif u < 6:
          swait((slg - 4) % NRS, bg)
          rwait(kg, slg)
          gstart(slg, bg)
        else:
          @pl.when(kg < KCH)
          def _():
            swait((slg - 4) % NRS, bg)
            rwait(kg, slg)
            gstart(slg, bg)

        gwait(u, b)
        scale(u, b)
        sstart(u, b)
      return c2

    lax.fori_loop(0, KCH // NRS, edge_oct, 0)
    for u in range(NRS - 4, NRS):
      swait(u, u % NBUF)
    plsc.subcore_barrier()

    for b in range(NBLK):
      rs = r0 + b * RB
      pltpu.sync_copy(acc_sp.at[pl.ds(rs, RB)], rows0)
      pltpu.sync_copy(rows0, h_sp.at[pl.ds(rs, RB)])
      pltpu.sync_copy(h0s_hbm.at[cid, pl.ds(rs, RB)],
                      acc_sp.at[pl.ds(rs, RB)])

    plsc.subcore_barrier()
    return carry

  lax.fori_loop(0, N_PROP, iteration, 0)

  pltpu.sync_copy(h_sp.at[pl.ds(r0, RPT)], out_hbm.at[cid, pl.ds(r0, RPT)])


@jax.jit
def _pagerank(logits_split, h0s, pack):
  mesh = plsc.VectorSubcoreMesh(
      core_axis_name="c", subcore_axis_name="s", num_cores=NC, num_subcores=NS)
  return pl.kernel(
      _body,
      out_type=jax.ShapeDtypeStruct((NC, N_PAD, DH), jnp.float32),
      mesh=mesh,
      compiler_params=pltpu.CompilerParams(use_tc_tiling_on_sc=False, needs_layout_passes=False),
      scratch_types=[
          pltpu.VMEM((NRS, 3, CB), jnp.int32),
          pltpu.VMEM((CB, DH), jnp.float32),
          pltpu.VMEM((CB, DH), jnp.float32),
          pltpu.VMEM((CB, DH), jnp.float32),
          pltpu.VMEM((CB, DH), jnp.float32),
          pltpu.VMEM_SHARED((N_PAD, DH), jnp.float32),
          pltpu.VMEM_SHARED((N_PAD, DH), jnp.float32),
      ] + [pltpu.SemaphoreType.DMA] * 16,
  )(logits_split, h0s, pack)


def kernel(logits, edge_index, edge_weight):
  dst = edge_index[0]
  src = edge_index[1]
  pad = E_PAD - E
  src_r = jnp.concatenate([src, jnp.zeros((pad,), jnp.int32)])
  dst_r = jnp.concatenate([dst, jnp.zeros((pad,), jnp.int32)])
  w_r = jnp.concatenate(
      [(1.0 - ALPHA) * edge_weight, jnp.zeros((pad,), jnp.float32)])
  w_i = lax.bitcast_convert_type(w_r, jnp.int32)
  pack = jnp.stack(
      [src_r.reshape(NS, KCH, CB), dst_r.reshape(NS, KCH, CB),
       w_i.reshape(NS, KCH, CB)], axis=2)
  logits_split = logits.reshape(N, NC, DH).transpose(1, 0, 2)
  logits_split = jnp.pad(logits_split, ((0, 0), (0, N_PAD - N), (0, 0)))
  out = _pagerank(logits_split, ALPHA * logits_split, pack)
  return out[:, :N, :].transpose(1, 0, 2).reshape(N, D)


# speedup vs baseline: 1.1235x; 1.0791x over previous
import jax
import jax.numpy as jnp
from jax import lax
from jax.experimental import pallas as pl
from jax.experimental.pallas import tpu as pltpu
from jax.experimental.pallas import tpu_sc as plsc

N = 10000
N_PAD = 10240
E = 320000
D = 128
ALPHA = 0.15
N_PROP = 5

NC = 2
NS = 16
L = 16

DH = D // NC
EPT = 20480
E_PAD = NS * EPT
CB = 128
KCH = EPT // CB
NBUF = 4
RPT = N_PAD // NS
RB = 128
NBLK = RPT // RB
FV = DH // L


def _lane_bcast(v, j):
  return lax.gather(
      v, jnp.full((L, 1), j, jnp.int32),
      lax.GatherDimensionNumbers(
          offset_dims=(), collapsed_slice_dims=(0,), start_index_map=(0,)),
      (1,), mode=lax.GatherScatterMode.PROMISE_IN_BOUNDS)


def _body(logits_hbm, h0s_hbm, pack_hbm, w_hbm, out_hbm,
          ring, wring, sbuf, rows0, rows1, rows2, rows3, h_sp, acc_sp,
          r0s, r1s, r2s, r3s, g0, g1, g2, g3, s0, s1, s2, s3):
  cid = lax.axis_index("c")
  sid = lax.axis_index("s")
  r0 = sid * RPT
  bufs = [rows0, rows1, rows2, rows3]
  rsems = [r0s, r1s, r2s, r3s]
  gsems = [g0, g1, g2, g3]
  ssems = [s0, s1, s2, s3]

  def zero_buf(buf):
    def zbody(r, carry):
      for f in range(FV):
        buf[r, pl.ds(f * L, L)] = jnp.zeros((L,), jnp.float32)
      return carry
    lax.fori_loop(0, RB, zbody, 0)

  def rstart(k, sl):
    pltpu.async_copy(pack_hbm.at[sid, k], ring.at[sl], rsems[sl])
    pltpu.async_copy(w_hbm.at[sid, k], wring.at[sl], rsems[sl])

  def rwait(k, sl):
    pltpu.make_async_copy(pack_hbm.at[sid, k], ring.at[sl], rsems[sl]).wait()
    pltpu.make_async_copy(w_hbm.at[sid, k], wring.at[sl], rsems[sl]).wait()

  def gstart(sl):
    pltpu.async_copy(h_sp.at[ring.at[sl, 0]], bufs[sl], gsems[sl])

  def gwait(sl):
    pltpu.make_async_copy(h_sp.at[ring.at[sl, 0]], bufs[sl], gsems[sl]).wait()

  def sstart(sl):
    pltpu.async_copy(bufs[sl], acc_sp.at[sbuf.at[sl]], ssems[sl], add=True)

  def swait(sl):
    pltpu.make_async_copy(bufs[sl], acc_sp.at[sbuf.at[sl]], ssems[sl]).wait()

  def scale(b):
    buf = bufs[b]
    for g in range(CB // L):
      sbuf[b, pl.ds(g * L, L)] = ring[b, 1, pl.ds(g * L, L)]
      w16 = wring[b, pl.ds(g * L, L)]
      for j in range(L):
        e = g * L + j
        we = _lane_bcast(w16, j)
        for f in range(FV):
          buf[e, pl.ds(f * L, L)] = buf[e, pl.ds(f * L, L)] * we

  for b in range(NBLK):
    pltpu.sync_copy(logits_hbm.at[cid, pl.ds(r0 + b * RB, RB)], rows1)
    pltpu.sync_copy(rows1, h_sp.at[pl.ds(r0 + b * RB, RB)])
    pltpu.sync_copy(h0s_hbm.at[cid, pl.ds(r0 + b * RB, RB)],
                    acc_sp.at[pl.ds(r0 + b * RB, RB)])

  plsc.subcore_barrier()

  def iteration(_, carry):
    for i in range(NBUF - 1):
      rstart(i, i)
    for i in range(2):
      rwait(i, i)
      gstart(i)

    def edge_quad(kk, c2):
      for b in range(NBUF):
        k = NBUF * kk + b
        kp = k + 3
        slp = (b + 3) % NBUF
        if b == 0:
          rstart(kp, slp)
        else:
          @pl.when(kp < KCH)
          def _():
            rstart(kp, slp)

        kg = k + 2
        slg = (b + 2) % NBUF
        if b < 2:
          @pl.when(kk > 0)
          def _():
            swait(slg)
          rwait(kg, slg)
          gstart(slg)
        else:
          @pl.when(kg < KCH)
          def _():
            swait(slg)
            rwait(kg, slg)
            gstart(slg)

        gwait(b)
        scale(b)
        sstart(b)
      return c2

    lax.fori_loop(0, KCH // NBUF, edge_quad, 0)
    for b in range(NBUF):
      swait(b)
    plsc.subcore_barrier()

    for b in range(NBLK):
      rs = r0 + b * RB
      pltpu.sync_copy(acc_sp.at[pl.ds(rs, RB)], rows0)
      pltpu.sync_copy(rows0, h_sp.at[pl.ds(rs, RB)])
      pltpu.sync_copy(h0s_hbm.at[cid, pl.ds(rs, RB)],
                      acc_sp.at[pl.ds(rs, RB)])

    plsc.subcore_barrier()
    return carry

  lax.fori_loop(0, N_PROP, iteration, 0)

  pltpu.sync_copy(h_sp.at[pl.ds(r0, RPT)], out_hbm.at[cid, pl.ds(r0, RPT)])


@jax.jit
def _pagerank(logits_split, h0s, pack, w):
  mesh = plsc.VectorSubcoreMesh(
      core_axis_name="c", subcore_axis_name="s", num_cores=NC, num_subcores=NS)
  return pl.kernel(
      _body,
      out_type=jax.ShapeDtypeStruct((NC, N_PAD, DH), jnp.float32),
      mesh=mesh,
      compiler_params=pltpu.CompilerParams(use_tc_tiling_on_sc=False),
      scratch_types=[
          pltpu.VMEM((NBUF, 2, CB), jnp.int32),
          pltpu.VMEM((NBUF, CB), jnp.float32),
          pltpu.VMEM((NBUF, CB), jnp.int32),
          pltpu.VMEM((CB, DH), jnp.float32),
          pltpu.VMEM((CB, DH), jnp.float32),
          pltpu.VMEM((CB, DH), jnp.float32),
          pltpu.VMEM((CB, DH), jnp.float32),
          pltpu.VMEM_SHARED((N_PAD, DH), jnp.float32),
          pltpu.VMEM_SHARED((N_PAD, DH), jnp.float32),
      ] + [pltpu.SemaphoreType.DMA] * 12,
  )(logits_split, h0s, pack, w)


def kernel(logits, edge_index, edge_weight):
  dst = edge_index[0]
  src = edge_index[1]
  pad = E_PAD - E
  src_r = jnp.concatenate([src, jnp.zeros((pad,), jnp.int32)])
  dst_r = jnp.concatenate([dst, jnp.zeros((pad,), jnp.int32)])
  w_r = jnp.concatenate(
      [(1.0 - ALPHA) * edge_weight, jnp.zeros((pad,), jnp.float32)])
  pack = jnp.stack(
      [src_r.reshape(NS, KCH, CB), dst_r.reshape(NS, KCH, CB)], axis=2)
  w = w_r.reshape(NS, KCH, CB)
  logits_split = logits.reshape(N, NC, DH).transpose(1, 0, 2)
  logits_split = jnp.pad(logits_split, ((0, 0), (0, N_PAD - N), (0, 0)))
  out = _pagerank(logits_split, ALPHA * logits_split, pack, w)
  return out[:, :N, :].transpose(1, 0, 2).reshape(N, D)
